# baseline (speedup 1.0000x reference)
import functools
import jax
import jax.numpy as jnp
from jax import lax
from jax.experimental import pallas as pl
from jax.experimental.pallas import tpu as pltpu, tpu_sc as plsc

_P = 10752
_GP = 21

_TCP = 8736
_BP = 336
_GRID = _TCP // _BP

_SCP = _P - _TCP
_NW = 32
_PW = _SCP // _NW
_NG = _PW // _GP

_C = (1.0 / 9.0, 1.0 / 7.0, 1.0 / 5.0, 1.0 / 3.0, 1.0)


def _log1p(u):
    z = u / (2.0 + u)
    z2 = z * z
    p = _C[0]
    for c in _C[1:]:
        p = p * z2 + c
    return 2.0 * z * p


def _tc_body(out_ref, lab_ref, res_ref, acc_ref):
    pid = pl.program_id(0)

    @pl.when(pid == 0)
    def _init():
        acc_ref[...] = jnp.zeros_like(acc_ref)

    z = jnp.zeros((32, 32), jnp.float32)
    part = [z] * 12

    for g in range(_BP // _GP):
        for c in range(3):
            p0 = _GP * g + 7 * c
            conf = lab_ref[p0]
            o0 = out_ref[p0]
            pos = jnp.maximum(conf, 0.0)
            neg = pos - conf
            a = jnp.abs(o0)
            base = jnp.log1p(jnp.exp(-a))
            r = jnp.maximum(o0, 0.0)
            part[0] += pos * (base + (a - r))
            part[1] += neg * (base + r)
            part[2] += pos
            part[3] += neg
            ge = o0 >= 0.0
            part[4] += jnp.where(ge, pos, 0.0)
            part[5] += jnp.where(ge, 0.0, neg)
            for f in range(1, 7):
                d = out_ref[p0 + f] - lab_ref[p0 + f]
                ad = jnp.abs(d)
                m = jnp.minimum(ad, 1.0)
                part[5 + f] += pos * (m * (ad - 0.5 * m))

    for q in range(12):
        acc_ref[q] += part[q]

    @pl.when(pid == _GRID - 1)
    def _final():
        for q in range(12):
            res_ref[q] = jnp.sum(acc_ref[q])


def _tc_partials(o3, t3):
    return pl.pallas_call(
        _tc_body,
        grid=(_GRID,),
        in_specs=[
            pl.BlockSpec((_BP, 32, 32), lambda i: (i, 0, 0)),
            pl.BlockSpec((_BP, 32, 32), lambda i: (i, 0, 0)),
        ],
        out_specs=pl.BlockSpec(memory_space=pltpu.SMEM),
        out_shape=jax.ShapeDtypeStruct((12,), jnp.float32),
        scratch_shapes=[pltpu.VMEM((12, 32, 32), jnp.float32)],
        compiler_params=pltpu.CompilerParams(
            dimension_semantics=("arbitrary",)),
    )(o3, t3)


def _sc_partials(o3, t3):
    mesh = plsc.VectorSubcoreMesh(core_axis_name="c", subcore_axis_name="s",
                                  num_cores=2, num_subcores=16)

    @functools.partial(
        pl.kernel, mesh=mesh,
        out_type=jax.ShapeDtypeStruct((_NW, 12, 16), jnp.float32),
        scratch_types=[
            pltpu.VMEM((_GP, 32, 32), jnp.float32),
            pltpu.VMEM((_GP, 32, 32), jnp.float32),
            pltpu.VMEM((_GP, 32, 32), jnp.float32),
            pltpu.VMEM((_GP, 32, 32), jnp.float32),
            pltpu.VMEM((12, 16), jnp.float32),
            pltpu.SemaphoreType.DMA,
            pltpu.SemaphoreType.DMA,
        ],
        compiler_params=pltpu.CompilerParams(use_tc_tiling_on_sc=False),
    )
    def k(o_hbm, t_hbm, res_hbm, obuf0, obuf1, tbuf0, tbuf1, pbuf, sem0, sem1):
        wid = lax.axis_index("s") * 2 + lax.axis_index("c")
        base = _TCP + wid * _PW
        sems = (sem0, sem1)
        obufs = (obuf0, obuf1)
        tbufs = (tbuf0, tbuf1)

        def fire(g, slot):
            p0 = base + g * _GP
            pltpu.async_copy(o_hbm.at[pl.ds(p0, _GP)], obufs[slot], sems[slot])
            pltpu.async_copy(t_hbm.at[pl.ds(p0, _GP)], tbufs[slot], sems[slot])

        def drain(slot):
            pltpu.make_async_copy(o_hbm.at[pl.ds(0, _GP)], obufs[slot],
                                  sems[slot]).wait()
            pltpu.make_async_copy(t_hbm.at[pl.ds(0, _GP)], tbufs[slot],
                                  sems[slot]).wait()

        def compute_group(slot, accs):
            obuf = obufs[slot]
            tbuf = tbufs[slot]

            def row_body(r, accs2):
                accs2 = list(accs2)
                for c in range(3):
                    p0 = 7 * c
                    for h in range(2):
                        cs = pl.ds(16 * h, 16)
                        conf = tbuf[p0, r, cs]
                        o0 = obuf[p0, r, cs]
                        pos = jnp.maximum(conf, 0.0)
                        neg = pos - conf
                        a = jnp.abs(o0)
                        base_sp = _log1p(jnp.exp(-a))
                        rl = jnp.maximum(o0, 0.0)
                        accs2[0] += pos * (base_sp + (a - rl))
                        accs2[1] += neg * (base_sp + rl)
                        accs2[2] += pos
                        accs2[3] += neg
                        ge = o0 >= 0.0
                        accs2[4] += jnp.where(ge, pos, 0.0)
                        accs2[5] += neg - jnp.where(ge, neg, 0.0)
                        for f in range(1, 7):
                            d = obuf[p0 + f, r, cs] - tbuf[p0 + f, r, cs]
                            ad = jnp.abs(d)
                            m = jnp.minimum(ad, 1.0)
                            accs2[5 + f] += pos * (m * (ad - 0.5 * m))
                return tuple(accs2)

            return lax.fori_loop(0, 32, row_body, accs)

        zero = jnp.zeros((16,), jnp.float32)
        accs = (zero,) * 12
        fire(0, 0)
        for g in range(_NG):
            slot = g % 2
            drain(slot)
            if g + 1 < _NG:
                fire(g + 1, 1 - slot)
            accs = compute_group(slot, accs)
        for i in range(12):
            pbuf[i] = accs[i]
        pltpu.sync_copy(pbuf, res_hbm.at[wid])

    return k(o3, t3)


def _combine_body(tc_ref, p_ref, res_ref):
    sums = [tc_ref[q] + jnp.sum(p_ref[:, q, :]) for q in range(12)]
    pb, nb, n_pos, n_neg, pc, nc = sums[:6]
    classify = 0.5 * pb / n_pos + 0.5 * nb / n_neg
    regs = [sums[5 + f] / n_pos for f in range(1, 7)]
    loss = classify
    for rv in regs:
        loss = loss + rv
    vals = [loss, classify] + regs + [pc, n_pos, nc, n_neg]
    for i, v in enumerate(vals):
        res_ref[i] = v


def kernel(output, labels):
    o3 = output.transpose(0, 1, 4, 5, 2, 3).reshape(_P, 32, 32)
    t3 = labels.transpose(0, 1, 4, 5, 2, 3).reshape(_P, 32, 32)
    sc = _sc_partials(o3, t3)
    tc = _tc_partials(o3, t3)
    res = pl.pallas_call(
        _combine_body,
        in_specs=[
            pl.BlockSpec(memory_space=pltpu.SMEM),
            pl.BlockSpec(memory_space=pltpu.VMEM),
        ],
        out_specs=pl.BlockSpec(memory_space=pltpu.SMEM),
        out_shape=jax.ShapeDtypeStruct((12,), jnp.float32),
    )(tc, sc)
    return tuple(res[i] for i in range(12))


# speedup vs baseline: 3.1652x; 3.1652x over previous
import jax
import jax.numpy as jnp
from jax.experimental import pallas as pl
from jax.experimental.pallas import tpu as pltpu

_PLANES = 16 * 32 * 3 * 7
_GROUPS = _PLANES // 21
_BG = 32
_BP = 21 * _BG
_GRID = _PLANES // _BP


def _body(out_ref, lab_ref, res_ref, acc_ref):
    pid = pl.program_id(0)

    @pl.when(pid == 0)
    def _init():
        acc_ref[...] = jnp.zeros_like(acc_ref)

    z = jnp.zeros((32, 32), jnp.float32)
    part = [z] * 12

    for g in range(_BG):
        for c in range(3):
            p0 = 21 * g + 7 * c
            conf = lab_ref[p0]
            o0 = out_ref[p0]
            pos = jnp.where(conf > 0.5, 1.0, 0.0)
            neg = jnp.where(conf < -0.5, 1.0, 0.0)
            a = jnp.abs(o0)
            base = jnp.log1p(jnp.exp(-a))
            r = jnp.maximum(o0, 0.0)
            part[0] += pos * (base + (a - r))
            part[1] += neg * (base + r)
            part[2] += pos
            part[3] += neg
            ge = o0 >= 0.0
            part[4] += jnp.where(ge, pos, 0.0)
            part[5] += jnp.where(ge, 0.0, neg)
            for f in range(1, 7):
                d = out_ref[p0 + f] - lab_ref[p0 + f]
                ad = jnp.abs(d)
                m = jnp.minimum(ad, 1.0)
                part[5 + f] += pos * (m * (ad - 0.5 * m))

    for q in range(12):
        acc_ref[q] += part[q]

    @pl.when(pid == _GRID - 1)
    def _final():
        sums = [jnp.sum(acc_ref[q]) for q in range(12)]
        pb, nb, n_pos, n_neg, pc, nc = sums[:6]
        classify = 0.5 * pb / n_pos + 0.5 * nb / n_neg
        regs = [sums[5 + f] / n_pos for f in range(1, 7)]
        loss = classify
        for rv in regs:
            loss = loss + rv
        vals = [loss, classify] + regs + [pc, n_pos, nc, n_neg]
        for i, v in enumerate(vals):
            res_ref[i] = v


def kernel(output, labels):
    o3 = output.transpose(0, 1, 4, 5, 2, 3).reshape(_PLANES, 32, 32)
    t3 = labels.transpose(0, 1, 4, 5, 2, 3).reshape(_PLANES, 32, 32)
    res = pl.pallas_call(
        _body,
        grid=(_GRID,),
        in_specs=[
            pl.BlockSpec((_BP, 32, 32), lambda i: (i, 0, 0)),
            pl.BlockSpec((_BP, 32, 32), lambda i: (i, 0, 0)),
        ],
        out_specs=pl.BlockSpec(memory_space=pltpu.SMEM),
        out_shape=jax.ShapeDtypeStruct((12,), jnp.float32),
        scratch_shapes=[pltpu.VMEM((12, 32, 32), jnp.float32)],
        compiler_params=pltpu.CompilerParams(
            dimension_semantics=("arbitrary",)),
    )(o3, t3)
    return tuple(res[i] for i in range(12))
